# baseline (speedup 1.0000x reference)
import jax
import jax.numpy as jnp
from jax import lax
from jax.experimental import pallas as pl
from jax.experimental.pallas import tpu as pltpu
from jax.experimental.pallas import tpu_sc as plsc

_NC = 2
_NS = 16
_NW = _NC * _NS

_N = 10000
_NPAD = 10240
_E = 320000
_D = 128
_EW = _E // _NW
_RPT = _NPAD // _NS

_DWIN = 100
_DNWIN = _EW // _DWIN

_WIN = 100
_NWIN = _EW // _WIN
_ICH = 25


def _deg_body(dst_hbm, out_hbm, idx_v, val_v, zero_v, acc_sh):
    c = lax.axis_index("c")
    s = lax.axis_index("s")
    wid = c * _NS + s
    for i in range(8):
        val_v[pl.ds(16 * i, 16)] = jnp.ones((16,), jnp.float32)
    for i in range(40):
        zero_v[pl.ds(16 * i, 16)] = jnp.zeros((16,), jnp.float32)
    pltpu.sync_copy(zero_v, acc_sh.at[pl.ds(s * _RPT, _RPT)])
    plsc.subcore_barrier()
    pltpu.sync_copy(dst_hbm.at[wid], idx_v)

    def win(j, carry):
        pltpu.sync_copy(val_v.at[pl.ds(0, _DWIN)],
                        acc_sh.at[idx_v.at[j]], add=True)
        return carry

    lax.fori_loop(0, _DNWIN, win, 0)
    plsc.subcore_barrier()
    pltpu.sync_copy(acc_sh.at[pl.ds(s * _RPT, _RPT)],
                    out_hbm.at[c, pl.ds(s * _RPT, _RPT)])


def _agg_body(hp_hbm, src_hbm, dst_hbm, out_hbm,
              sidx_v, didx_v, r0, r1, r2, acc_sh,
              g0, g1, g2, s0, s1, s2):
    c = lax.axis_index("c")
    s = lax.axis_index("s")
    wid = c * _NS + s

    pltpu.async_copy(src_hbm.at[wid, 0], sidx_v, g0)
    pltpu.async_copy(dst_hbm.at[wid, 0], didx_v, g1)

    def zrow(i, carry):
        for k in range(8):
            r0[i, pl.ds(16 * k, 16)] = jnp.zeros((16,), jnp.float32)
        return carry

    lax.fori_loop(0, _WIN, zrow, 0)

    @pl.when(jnp.logical_and(c == 0, s < _NS - 1))
    def _():
        pltpu.sync_copy(hp_hbm.at[pl.ds(s * _RPT, _RPT)],
                        acc_sh.at[pl.ds(s * _RPT, _RPT)])

    @pl.when(jnp.logical_and(c == 0, s == _NS - 1))
    def _():
        pltpu.sync_copy(hp_hbm.at[pl.ds(_N - 400, 400)],
                        acc_sh.at[pl.ds(_N - 400, 400)])
        pltpu.sync_copy(r0, acc_sh.at[pl.ds(_N, _WIN)])
        pltpu.sync_copy(r0, acc_sh.at[pl.ds(_N + _WIN, _WIN)])
        pltpu.sync_copy(r0.at[pl.ds(0, 40)],
                        acc_sh.at[pl.ds(_N + 2 * _WIN, 40)])

    @pl.when(c == 1)
    def _():
        for t in range(6):
            pltpu.sync_copy(r0,
                            acc_sh.at[pl.ds(s * _RPT + t * _WIN, _WIN)])
        pltpu.sync_copy(r0.at[pl.ds(0, _RPT - 6 * _WIN)],
                        acc_sh.at[pl.ds(s * _RPT + 6 * _WIN,
                                        _RPT - 6 * _WIN)])

    plsc.subcore_barrier()

    rbufs = (r0, r1, r2)
    gsems = (g0, g1, g2)
    ssems = (s0, s1, s2)

    def gst(w, b):
        pltpu.async_copy(hp_hbm.at[sidx_v.at[w]], rbufs[b], gsems[b])

    def gwt(w, b):
        pltpu.make_async_copy(hp_hbm.at[sidx_v.at[w]], rbufs[b],
                              gsems[b]).wait()

    def sst(w, b):
        pltpu.async_copy(rbufs[b], acc_sh.at[didx_v.at[w]], ssems[b],
                         add=True)

    def swt(w, b):
        pltpu.make_async_copy(rbufs[b], acc_sh.at[didx_v.at[w]],
                              ssems[b]).wait()

    for ch in range(_NWIN // _ICH):
        if ch == 0:
            pltpu.make_async_copy(src_hbm.at[wid, 0], sidx_v, g0).wait()
            pltpu.make_async_copy(dst_hbm.at[wid, 0], didx_v, g1).wait()
        else:
            pltpu.sync_copy(src_hbm.at[wid, ch], sidx_v)
            pltpu.sync_copy(dst_hbm.at[wid, ch], didx_v)
        gst(0, 0)
        gst(1, 1)

        def body(t, carry):
            w = 3 * t
            gwt(w, 0)
            sst(w, 0)

            @pl.when(t > 0)
            def _():
                swt(w - 1, 2)

            gst(w + 2, 2)

            gwt(w + 1, 1)
            sst(w + 1, 1)
            swt(w, 0)
            gst(w + 3, 0)

            gwt(w + 2, 2)
            sst(w + 2, 2)
            swt(w + 1, 1)
            gst(w + 4, 1)
            return carry

        lax.fori_loop(0, 7, body, 0)
        gwt(21, 0)
        sst(21, 0)
        swt(20, 2)
        gst(23, 2)
        gwt(22, 1)
        sst(22, 1)
        swt(21, 0)
        gst(24, 0)
        gwt(23, 2)
        sst(23, 2)
        swt(22, 1)
        gwt(24, 0)
        sst(24, 0)
        swt(23, 2)
        swt(24, 0)

    plsc.subcore_barrier()
    pltpu.sync_copy(acc_sh.at[pl.ds(s * _RPT, _RPT)],
                    out_hbm.at[c, pl.ds(s * _RPT, _RPT)])


def _tc_mm(x_ref, w_ref, h_ref):
    h_ref[...] = lax.dot_general(
        x_ref[...], w_ref[...], (((1,), (1,)), ((), ())),
        preferred_element_type=jnp.float32)


def _tc_norm(h_ref, d_ref, hp_ref, dinv_ref):
    deg = d_ref[0] + d_ref[1] + 1.0
    dinv = lax.rsqrt(deg)
    hp_ref[...] = h_ref[...] * dinv
    dinv_ref[...] = dinv


def _tc_mid(p_ref, dinv_ref, b_ref, w_ref, out_ref):
    dinv = dinv_ref[...]
    z = dinv * (p_ref[0] + p_ref[1]) + b_ref[...]
    y = jnp.maximum(z, 0.0)
    h = lax.dot_general(y, w_ref[...], (((1,), (1,)), ((), ())),
                        preferred_element_type=jnp.float32)
    out_ref[...] = h * dinv


def _tc_last(p_ref, dinv_ref, b_ref, out_ref):
    z = dinv_ref[...] * (p_ref[0] + p_ref[1]) + b_ref[...]
    out_ref[...] = z


_BLK = 2000
_GRID = _N // _BLK

_row = lambda i: (i, 0)
_rep = lambda i: (0, 0)
_fspec = pl.BlockSpec((_BLK, _D), _row)
_cspec = pl.BlockSpec((_BLK, 1), _row)
_wspec = pl.BlockSpec((_D, _D), _rep)
_bspec = pl.BlockSpec((1, _D), _rep)
_pspec = pl.BlockSpec((2, _BLK, _D), lambda i: (0, i, 0))
_dspec = pl.BlockSpec((2, _BLK, 1), lambda i: (0, i, 0))
_fshape = jax.ShapeDtypeStruct((_N, _D), jnp.float32)
_cshape = jax.ShapeDtypeStruct((_N, 1), jnp.float32)


def kernel(x, edge_index, W1, b1, W2, b2, W3, b3):
    src = edge_index[0].reshape(_NW, _NWIN // _ICH, _ICH, _WIN)
    dst = edge_index[1].reshape(_NW, _NWIN // _ICH, _ICH, _WIN)
    dstd = edge_index[1].reshape(_NW, _DNWIN, _DWIN)

    mesh = plsc.VectorSubcoreMesh(core_axis_name="c", subcore_axis_name="s",
                                  num_cores=_NC, num_subcores=_NS)

    deg_call = pl.kernel(
        _deg_body,
        out_type=jax.ShapeDtypeStruct((_NC, _NPAD), jnp.float32),
        mesh=mesh,
        scratch_types=[
            pltpu.VMEM((_DNWIN, _DWIN), jnp.int32),
            pltpu.VMEM((128,), jnp.float32),
            pltpu.VMEM((_RPT,), jnp.float32),
            pltpu.VMEM_SHARED((_NPAD,), jnp.float32),
        ],
    )
    degp = deg_call(dstd)

    agg_call = pl.kernel(
        _agg_body,
        out_type=jax.ShapeDtypeStruct((_NC, _NPAD, _D), jnp.float32),
        mesh=mesh,
        scratch_types=[
            pltpu.VMEM((_ICH, _WIN), jnp.int32),
            pltpu.VMEM((_ICH, _WIN), jnp.int32),
            pltpu.VMEM((_WIN, _D), jnp.float32),
            pltpu.VMEM((_WIN, _D), jnp.float32),
            pltpu.VMEM((_WIN, _D), jnp.float32),
            pltpu.VMEM_SHARED((_NPAD, _D), jnp.float32),
            pltpu.SemaphoreType.DMA,
            pltpu.SemaphoreType.DMA,
            pltpu.SemaphoreType.DMA,
            pltpu.SemaphoreType.DMA,
            pltpu.SemaphoreType.DMA,
            pltpu.SemaphoreType.DMA,
        ],
    )

    dd = degp.reshape(2, _NPAD, 1)

    h1 = pl.pallas_call(
        _tc_mm,
        grid=(_GRID,),
        in_specs=[_fspec, _wspec],
        out_specs=_fspec,
        out_shape=_fshape,
    )(x, W1)

    hp1, dinv = pl.pallas_call(
        _tc_norm,
        grid=(_GRID,),
        in_specs=[_fspec, _dspec],
        out_specs=[_fspec, _cspec],
        out_shape=[_fshape, _cshape],
    )(h1, dd)

    p = agg_call(hp1, src, dst)
    hp2 = pl.pallas_call(
        _tc_mid,
        grid=(_GRID,),
        in_specs=[_pspec, _cspec, _bspec, _wspec],
        out_specs=_fspec,
        out_shape=_fshape,
    )(p, dinv, b1.reshape(1, _D), W2)

    p = agg_call(hp2, src, dst)
    hp3 = pl.pallas_call(
        _tc_mid,
        grid=(_GRID,),
        in_specs=[_pspec, _cspec, _bspec, _wspec],
        out_specs=_fspec,
        out_shape=_fshape,
    )(p, dinv, b2.reshape(1, _D), W3)

    p = agg_call(hp3, src, dst)
    out = pl.pallas_call(
        _tc_last,
        grid=(_GRID,),
        in_specs=[_pspec, _cspec, _bspec],
        out_specs=_fspec,
        out_shape=_fshape,
    )(p, dinv, b3.reshape(1, _D))
    return out


# speedup vs baseline: 1.0196x; 1.0196x over previous
import jax
import jax.numpy as jnp
from jax import lax
from jax.experimental import pallas as pl
from jax.experimental.pallas import tpu as pltpu
from jax.experimental.pallas import tpu_sc as plsc

_NC = 2
_NS = 16
_NW = _NC * _NS

_N = 10000
_NPAD = 10240
_E = 320000
_D = 128
_EW = _E // _NW
_RPT = _NPAD // _NS

_DWIN = 100
_DNWIN = _EW // _DWIN

_WIN = 100
_NWIN = _EW // _WIN
_ICH = 25


def _deg_body(dst_hbm, out_hbm, idx_v, val_v, zero_v, acc_sh):
    c = lax.axis_index("c")
    s = lax.axis_index("s")
    wid = c * _NS + s
    for i in range(8):
        val_v[pl.ds(16 * i, 16)] = jnp.ones((16,), jnp.float32)
    for i in range(40):
        zero_v[pl.ds(16 * i, 16)] = jnp.zeros((16,), jnp.float32)
    pltpu.sync_copy(zero_v, acc_sh.at[pl.ds(s * _RPT, _RPT)])
    plsc.subcore_barrier()
    pltpu.sync_copy(dst_hbm.at[wid], idx_v)

    def win(j, carry):
        pltpu.sync_copy(val_v.at[pl.ds(0, _DWIN)],
                        acc_sh.at[idx_v.at[j]], add=True)
        return carry

    lax.fori_loop(0, _DNWIN, win, 0)
    plsc.subcore_barrier()
    pltpu.sync_copy(acc_sh.at[pl.ds(s * _RPT, _RPT)],
                    out_hbm.at[c, pl.ds(s * _RPT, _RPT)])


def _agg_body(hp_hbm, src_hbm, dst_hbm, out_hbm,
              sidx_v, didx_v, r0, r1, r2, acc_sh,
              g0, g1, g2, s0, s1, s2):
    c = lax.axis_index("c")
    s = lax.axis_index("s")
    wid = c * _NS + s

    pltpu.async_copy(src_hbm.at[wid, 0], sidx_v, g0)
    pltpu.async_copy(dst_hbm.at[wid, 0], didx_v, g1)

    def zrow(i, carry):
        for k in range(8):
            r0[i, pl.ds(16 * k, 16)] = jnp.zeros((16,), jnp.float32)
        return carry

    lax.fori_loop(0, _WIN, zrow, 0)
    for t in range(6):
        pltpu.sync_copy(r0, acc_sh.at[pl.ds(s * _RPT + t * _WIN, _WIN)])
    pltpu.sync_copy(r0.at[pl.ds(0, _RPT - 6 * _WIN)],
                    acc_sh.at[pl.ds(s * _RPT + 6 * _WIN, _RPT - 6 * _WIN)])
    plsc.subcore_barrier()

    rbufs = (r0, r1, r2)
    gsems = (g0, g1, g2)
    ssems = (s0, s1, s2)

    def gst(w, b):
        pltpu.async_copy(hp_hbm.at[sidx_v.at[w]], rbufs[b], gsems[b])

    def gwt(w, b):
        pltpu.make_async_copy(hp_hbm.at[sidx_v.at[w]], rbufs[b],
                              gsems[b]).wait()

    def sst(w, b):
        pltpu.async_copy(rbufs[b], acc_sh.at[didx_v.at[w]], ssems[b],
                         add=True)

    def swt(w, b):
        pltpu.make_async_copy(rbufs[b], acc_sh.at[didx_v.at[w]],
                              ssems[b]).wait()

    for ch in range(_NWIN // _ICH):
        if ch == 0:
            pltpu.make_async_copy(src_hbm.at[wid, 0], sidx_v, g0).wait()
            pltpu.make_async_copy(dst_hbm.at[wid, 0], didx_v, g1).wait()
        else:
            pltpu.sync_copy(src_hbm.at[wid, ch], sidx_v)
            pltpu.sync_copy(dst_hbm.at[wid, ch], didx_v)
        gst(0, 0)
        gst(1, 1)

        def body(t, carry):
            w = 3 * t
            gwt(w, 0)
            sst(w, 0)

            @pl.when(t > 0)
            def _():
                swt(w - 1, 2)

            gst(w + 2, 2)

            gwt(w + 1, 1)
            sst(w + 1, 1)
            swt(w, 0)
            gst(w + 3, 0)

            gwt(w + 2, 2)
            sst(w + 2, 2)
            swt(w + 1, 1)
            gst(w + 4, 1)
            return carry

        lax.fori_loop(0, 7, body, 0)
        gwt(21, 0)
        sst(21, 0)
        swt(20, 2)
        gst(23, 2)
        gwt(22, 1)
        sst(22, 1)
        swt(21, 0)
        gst(24, 0)
        gwt(23, 2)
        sst(23, 2)
        swt(22, 1)
        gwt(24, 0)
        sst(24, 0)
        swt(23, 2)
        swt(24, 0)

    plsc.subcore_barrier()
    pltpu.sync_copy(acc_sh.at[pl.ds(s * _RPT, _RPT)],
                    out_hbm.at[c, pl.ds(s * _RPT, _RPT)])


def _tc_mm(x_ref, w_ref, h_ref):
    h_ref[...] = lax.dot_general(
        x_ref[...], w_ref[...], (((1,), (1,)), ((), ())),
        preferred_element_type=jnp.float32)


def _tc_norm(h_ref, d_ref, hp_ref, dinv_ref):
    deg = d_ref[0] + d_ref[1] + 1.0
    dinv = lax.rsqrt(deg)
    hp_ref[...] = h_ref[...] * dinv
    dinv_ref[...] = dinv


def _tc_mid(p_ref, hp_ref, dinv_ref, b_ref, w_ref, out_ref):
    dinv = dinv_ref[...]
    z = dinv * (p_ref[0] + p_ref[1] + hp_ref[...]) + b_ref[...]
    y = jnp.maximum(z, 0.0)
    h = lax.dot_general(y, w_ref[...], (((1,), (1,)), ((), ())),
                        preferred_element_type=jnp.float32)
    out_ref[...] = h * dinv


def _tc_last(p_ref, hp_ref, dinv_ref, b_ref, out_ref):
    z = (dinv_ref[...] * (p_ref[0] + p_ref[1] + hp_ref[...])
         + b_ref[...])
    out_ref[...] = z


_BLK = 2000
_GRID = _N // _BLK

_row = lambda i: (i, 0)
_rep = lambda i: (0, 0)
_fspec = pl.BlockSpec((_BLK, _D), _row)
_cspec = pl.BlockSpec((_BLK, 1), _row)
_wspec = pl.BlockSpec((_D, _D), _rep)
_bspec = pl.BlockSpec((1, _D), _rep)
_pspec = pl.BlockSpec((2, _BLK, _D), lambda i: (0, i, 0))
_dspec = pl.BlockSpec((2, _BLK, 1), lambda i: (0, i, 0))
_fshape = jax.ShapeDtypeStruct((_N, _D), jnp.float32)
_cshape = jax.ShapeDtypeStruct((_N, 1), jnp.float32)


def kernel(x, edge_index, W1, b1, W2, b2, W3, b3):
    src = edge_index[0].reshape(_NW, _NWIN // _ICH, _ICH, _WIN)
    dst = edge_index[1].reshape(_NW, _NWIN // _ICH, _ICH, _WIN)
    dstd = edge_index[1].reshape(_NW, _DNWIN, _DWIN)

    mesh = plsc.VectorSubcoreMesh(core_axis_name="c", subcore_axis_name="s",
                                  num_cores=_NC, num_subcores=_NS)

    deg_call = pl.kernel(
        _deg_body,
        out_type=jax.ShapeDtypeStruct((_NC, _NPAD), jnp.float32),
        mesh=mesh,
        scratch_types=[
            pltpu.VMEM((_DNWIN, _DWIN), jnp.int32),
            pltpu.VMEM((128,), jnp.float32),
            pltpu.VMEM((_RPT,), jnp.float32),
            pltpu.VMEM_SHARED((_NPAD,), jnp.float32),
        ],
    )
    degp = deg_call(dstd)

    agg_call = pl.kernel(
        _agg_body,
        out_type=jax.ShapeDtypeStruct((_NC, _NPAD, _D), jnp.float32),
        mesh=mesh,
        scratch_types=[
            pltpu.VMEM((_ICH, _WIN), jnp.int32),
            pltpu.VMEM((_ICH, _WIN), jnp.int32),
            pltpu.VMEM((_WIN, _D), jnp.float32),
            pltpu.VMEM((_WIN, _D), jnp.float32),
            pltpu.VMEM((_WIN, _D), jnp.float32),
            pltpu.VMEM_SHARED((_NPAD, _D), jnp.float32),
            pltpu.SemaphoreType.DMA,
            pltpu.SemaphoreType.DMA,
            pltpu.SemaphoreType.DMA,
            pltpu.SemaphoreType.DMA,
            pltpu.SemaphoreType.DMA,
            pltpu.SemaphoreType.DMA,
        ],
    )

    dd = degp.reshape(2, _NPAD, 1)

    h1 = pl.pallas_call(
        _tc_mm,
        grid=(_GRID,),
        in_specs=[_fspec, _wspec],
        out_specs=_fspec,
        out_shape=_fshape,
    )(x, W1)

    hp1, dinv = pl.pallas_call(
        _tc_norm,
        grid=(_GRID,),
        in_specs=[_fspec, _dspec],
        out_specs=[_fspec, _cspec],
        out_shape=[_fshape, _cshape],
    )(h1, dd)

    p = agg_call(hp1, src, dst)
    hp2 = pl.pallas_call(
        _tc_mid,
        grid=(_GRID,),
        in_specs=[_pspec, _fspec, _cspec, _bspec, _wspec],
        out_specs=_fspec,
        out_shape=_fshape,
    )(p, hp1, dinv, b1.reshape(1, _D), W2)

    p = agg_call(hp2, src, dst)
    hp3 = pl.pallas_call(
        _tc_mid,
        grid=(_GRID,),
        in_specs=[_pspec, _fspec, _cspec, _bspec, _wspec],
        out_specs=_fspec,
        out_shape=_fshape,
    )(p, hp2, dinv, b2.reshape(1, _D), W3)

    p = agg_call(hp3, src, dst)
    out = pl.pallas_call(
        _tc_last,
        grid=(_GRID,),
        in_specs=[_pspec, _fspec, _cspec, _bspec],
        out_specs=_fspec,
        out_shape=_fshape,
    )(p, hp3, dinv, b3.reshape(1, _D))
    return out
